# baseline (speedup 1.0000x reference)
import functools

import jax
import jax.numpy as jnp
from jax import lax
from jax.experimental import pallas as pl
from jax.experimental.pallas import tpu as pltpu, tpu_sc as plsc

_INFO = plsc.get_sparse_core_info()
_NC = _INFO.num_cores
_NS = _INFO.num_subcores
_NW = _NC * _NS
_L = _INFO.num_lanes

_BATCH = 16384
_LD = 16
_BPW = _BATCH // _NW
_CHUNK = 128
_NCHUNK = _BPW // _CHUNK
_NGROUP = _BPW // _L


def _mf_body(uid_hbm, iid_hbm, uw_hbm, iw_hbm, bu_hbm, bi_hbm, bias_hbm,
             out_hbm,
             idx_u, idx_i, u_rows, i_rows, bu_v, bi_v, bias_v, out_v, sem):
    wid = lax.axis_index("s") * _NC + lax.axis_index("c")

    rbase = wid * _NCHUNK
    pltpu.sync_copy(uid_hbm.at[pl.ds(rbase, _NCHUNK)], idx_u)
    pltpu.sync_copy(iid_hbm.at[pl.ds(rbase, _NCHUNK)], idx_i)
    pltpu.sync_copy(bias_hbm, bias_v)

    copies = []
    for j in range(_NCHUNK):
        dst = pl.ds(j * _CHUNK, _CHUNK)
        copies.append(pltpu.async_copy(uw_hbm.at[idx_u.at[j]], u_rows.at[dst], sem))
        copies.append(pltpu.async_copy(iw_hbm.at[idx_i.at[j]], i_rows.at[dst], sem))
        copies.append(pltpu.async_copy(bu_hbm.at[idx_u.at[j]], bu_v.at[dst], sem))
        copies.append(pltpu.async_copy(bi_hbm.at[idx_i.at[j]], bi_v.at[dst], sem))
    for cp in copies:
        cp.wait()

    bias_vec = bias_v[...]

    def group(g, _):
        rows = g * _L + lax.iota(jnp.int32, _L)
        acc = bu_v[pl.ds(g * _L, _L)] + bi_v[pl.ds(g * _L, _L)] + bias_vec
        for d in range(_LD):
            dcol = jnp.full((_L,), d, jnp.int32)
            uv = plsc.load_gather(u_rows, [rows, dcol])
            iv = plsc.load_gather(i_rows, [rows, dcol])
            acc = acc + uv * iv
        out_v[pl.ds(g * _L, _L)] = acc
        return 0

    lax.fori_loop(0, _NGROUP, group, 0)
    pltpu.sync_copy(out_v, out_hbm.at[pl.ds(wid * _BPW, _BPW)])


@jax.jit
def _mf_sc(uid2d, iid2d, user_weight, item_weight, bias_user, bias_item, bias16):
    mesh = plsc.VectorSubcoreMesh(core_axis_name="c", subcore_axis_name="s")
    return pl.kernel(
        _mf_body,
        out_type=jax.ShapeDtypeStruct((_BATCH,), jnp.float32),
        mesh=mesh,
        compiler_params=pltpu.CompilerParams(needs_layout_passes=False,
                                             use_tc_tiling_on_sc=False),
        scratch_types=[
            pltpu.VMEM((_NCHUNK, _CHUNK), jnp.int32),
            pltpu.VMEM((_NCHUNK, _CHUNK), jnp.int32),
            pltpu.VMEM((_BPW, _LD), jnp.float32),
            pltpu.VMEM((_BPW, _LD), jnp.float32),
            pltpu.VMEM((_BPW,), jnp.float32),
            pltpu.VMEM((_BPW,), jnp.float32),
            pltpu.VMEM((_L,), jnp.float32),
            pltpu.VMEM((_BPW,), jnp.float32),
            pltpu.SemaphoreType.DMA,
        ],
    )(uid2d, iid2d, user_weight, item_weight, bias_user, bias_item, bias16)


def kernel(train_x, user_weight, item_weight, bias_user_weight, bias_item_weight, bias):
    uid2d = train_x[:, 0].reshape(_BATCH // _CHUNK, _CHUNK)
    iid2d = train_x[:, 1].reshape(_BATCH // _CHUNK, _CHUNK)
    bias16 = jnp.broadcast_to(bias, (_L,))
    return _mf_sc(uid2d, iid2d, user_weight, item_weight,
                  bias_user_weight.reshape(-1), bias_item_weight.reshape(-1),
                  bias16)


# speedup vs baseline: 3.5798x; 3.5798x over previous
import functools

import jax
import jax.numpy as jnp
from jax import lax
from jax.experimental import pallas as pl
from jax.experimental.pallas import tpu as pltpu, tpu_sc as plsc

_INFO = plsc.get_sparse_core_info()
_NC = _INFO.num_cores
_NS = _INFO.num_subcores
_NW = _NC * _NS
_L = _INFO.num_lanes

_N = 1000000
_BATCH = 16384
_LD = 16
_BPW = _BATCH // _NW
_CHUNK = 128
_NCHUNK = _BPW // _CHUNK
_G = 16
_NGROUP = _BPW // _G


def _bias_body(uid_hbm, iid_hbm, bu_hbm, bi_hbm, bias_hbm, out_hbm,
               idx_u, idx_i, bu_v, bi_v, bias_v, out_v, sem):
    wid = lax.axis_index("s") * _NC + lax.axis_index("c")
    base = wid * _BPW
    pltpu.sync_copy(uid_hbm.at[pl.ds(base, _BPW)], idx_u)
    pltpu.sync_copy(iid_hbm.at[pl.ds(base, _BPW)], idx_i)
    pltpu.sync_copy(bias_hbm, bias_v)
    copies = []
    for j in range(_NCHUNK):
        sl = pl.ds(j * _CHUNK, _CHUNK)
        copies.append(pltpu.async_copy(bu_hbm.at[idx_u.at[sl]], bu_v.at[sl], sem))
        copies.append(pltpu.async_copy(bi_hbm.at[idx_i.at[sl]], bi_v.at[sl], sem))
    for cp in copies:
        cp.wait()
    bias_vec = bias_v[...]

    def group(g, _):
        sl = pl.ds(g * _L, _L)
        out_v[sl] = bu_v[sl] + bi_v[sl] + bias_vec
        return 0

    lax.fori_loop(0, _BPW // _L, group, 0)
    pltpu.sync_copy(out_v, out_hbm.at[pl.ds(base, _BPW)])


@functools.partial(
    pl.kernel,
    out_type=jax.ShapeDtypeStruct((_BATCH,), jnp.float32),
    mesh=plsc.VectorSubcoreMesh(core_axis_name="c", subcore_axis_name="s"),
    scratch_types=[
        pltpu.VMEM((_BPW,), jnp.int32),
        pltpu.VMEM((_BPW,), jnp.int32),
        pltpu.VMEM((_BPW,), jnp.float32),
        pltpu.VMEM((_BPW,), jnp.float32),
        pltpu.VMEM((_L,), jnp.float32),
        pltpu.VMEM((_BPW,), jnp.float32),
        pltpu.SemaphoreType.DMA,
    ],
    compiler_params=pltpu.CompilerParams(needs_layout_passes=False,
                                         use_tc_tiling_on_sc=False),
)
def _bias_kernel(*args):
    _bias_body(*args)


def _main_body(uid_hbm, iid_hbm, uw3_hbm, iw3_hbm, bsum_hbm, out_hbm,
               idx_u, idx_i, bsum_v, u_chunks, i_chunks, out_v, sem):
    wid = lax.axis_index("s") * _NC + lax.axis_index("c")
    base = wid * _BPW
    pltpu.sync_copy(uid_hbm.at[pl.ds(base, _BPW)], idx_u)
    pltpu.sync_copy(iid_hbm.at[pl.ds(base, _BPW)], idx_i)
    pltpu.sync_copy(bsum_hbm.at[pl.ds(base, _BPW)], bsum_v)

    lane = lax.iota(jnp.int32, _L)

    def group(g, _):
        sl = pl.ds(g * _G, _G)
        uc = idx_u[sl]
        ic = idx_i[sl]
        ubase = (uc >> 7) << 7
        ibase = (ic >> 7) << 7
        copies = []
        for j in range(_G):
            cu = pl.multiple_of(ubase[j], 128)
            ci = pl.multiple_of(ibase[j], 128)
            copies.append(pltpu.async_copy(
                uw3_hbm.at[:, :, pl.ds(cu, 128)], u_chunks.at[j], sem))
            copies.append(pltpu.async_copy(
                iw3_hbm.at[:, :, pl.ds(ci, 128)], i_chunks.at[j], sem))
        for cp in copies:
            cp.wait()

        lu = uc & 127
        li = ic & 127
        acc = bsum_v[sl]
        for f in range(_LD):
            uv = plsc.load_gather(u_chunks, [lane, jnp.full((_L,), f // 8, jnp.int32),
                                             jnp.full((_L,), f % 8, jnp.int32), lu])
            iv = plsc.load_gather(i_chunks, [lane, jnp.full((_L,), f // 8, jnp.int32),
                                             jnp.full((_L,), f % 8, jnp.int32), li])
            acc = acc + uv * iv
        out_v[sl] = acc
        return 0

    lax.fori_loop(0, _NGROUP, group, 0)
    pltpu.sync_copy(out_v, out_hbm.at[pl.ds(base, _BPW)])


@functools.partial(
    pl.kernel,
    out_type=jax.ShapeDtypeStruct((_BATCH,), jnp.float32),
    mesh=plsc.VectorSubcoreMesh(core_axis_name="c", subcore_axis_name="s"),
    scratch_types=[
        pltpu.VMEM((_BPW,), jnp.int32),
        pltpu.VMEM((_BPW,), jnp.int32),
        pltpu.VMEM((_BPW,), jnp.float32),
        pltpu.VMEM((_G, 2, 8, 128), jnp.float32),
        pltpu.VMEM((_G, 2, 8, 128), jnp.float32),
        pltpu.VMEM((_BPW,), jnp.float32),
        pltpu.SemaphoreType.DMA,
    ],
    compiler_params=pltpu.CompilerParams(needs_layout_passes=False,
                                         use_tc_tiling_on_sc=True,
                                         disable_bounds_checks=True),
)
def _main_kernel(*args):
    _main_body(*args)


def kernel(train_x, user_weight, item_weight, bias_user_weight, bias_item_weight, bias):
    uid = train_x[:, 0]
    iid = train_x[:, 1]
    bias16 = jnp.broadcast_to(bias, (_L,))
    bsum = _bias_kernel(uid, iid, bias_user_weight.reshape(-1),
                        bias_item_weight.reshape(-1), bias16)
    uw3 = user_weight.T.reshape(2, 8, _N)
    iw3 = item_weight.T.reshape(2, 8, _N)
    return _main_kernel(uid, iid, uw3, iw3, bsum)


# speedup vs baseline: 3.6687x; 1.0248x over previous
import functools

import jax
import jax.numpy as jnp
from jax import lax
from jax.experimental import pallas as pl
from jax.experimental.pallas import tpu as pltpu, tpu_sc as plsc

_INFO = plsc.get_sparse_core_info()
_NC = _INFO.num_cores
_NS = _INFO.num_subcores
_NW = _NC * _NS
_L = _INFO.num_lanes

_N = 1000000
_BATCH = 16384
_LD = 16
_BPW = _BATCH // _NW
_CHUNK = 128
_NCHUNK = _BPW // _CHUNK
_G = 8
_NGROUP = _BPW // _G
_PAD = _BPW + _L


def _fire(g, slot, idx_u, idx_i, uw3_hbm, iw3_hbm, u_bufs, i_bufs, sems):
    uc = idx_u[pl.ds(g * _G, _L)]
    ic = idx_i[pl.ds(g * _G, _L)]
    ub = (uc >> 7) << 7
    ib = (ic >> 7) << 7
    for j in range(_G):
        cu = pl.multiple_of(ub[j], 128)
        ci = pl.multiple_of(ib[j], 128)
        pltpu.async_copy(uw3_hbm.at[:, :, pl.ds(cu, 128)],
                         u_bufs.at[slot, j], sems.at[slot])
        pltpu.async_copy(iw3_hbm.at[:, :, pl.ds(ci, 128)],
                         i_bufs.at[slot, j], sems.at[slot])


def _main_body(uid_hbm, iid_hbm, uw3_hbm, iw3_hbm, bu_hbm, bi_hbm, bias_hbm,
               out_hbm,
               idx_u, idx_i, bu_v, bi_v, bias_v, out_v, u_bufs, i_bufs,
               sem_b, sems):
    wid = lax.axis_index("s") * _NC + lax.axis_index("c")
    base = wid * _BPW
    pltpu.sync_copy(uid_hbm.at[pl.ds(base, _BPW)], idx_u.at[pl.ds(0, _BPW)])
    pltpu.sync_copy(iid_hbm.at[pl.ds(base, _BPW)], idx_i.at[pl.ds(0, _BPW)])
    pltpu.sync_copy(bias_hbm, bias_v)

    bias_copies = []
    for j in range(_NCHUNK):
        sl = pl.ds(j * _CHUNK, _CHUNK)
        bias_copies.append(pltpu.async_copy(bu_hbm.at[idx_u.at[sl]], bu_v.at[sl], sem_b))
        bias_copies.append(pltpu.async_copy(bi_hbm.at[idx_i.at[sl]], bi_v.at[sl], sem_b))

    _fire(0, 0, idx_u, idx_i, uw3_hbm, iw3_hbm, u_bufs, i_bufs, sems)
    for cp in bias_copies:
        cp.wait()

    bias_vec = bias_v[...]
    lane = lax.iota(jnp.int32, _L)
    elem = lane & (_G - 1)

    def group(g, _):
        slot = g & 1

        @pl.when(g + 1 < _NGROUP)
        def _():
            _fire(g + 1, slot ^ 1, idx_u, idx_i, uw3_hbm, iw3_hbm,
                  u_bufs, i_bufs, sems)

        for j in range(_G):
            pltpu.make_async_copy(uw3_hbm.at[:, :, pl.ds(0, 128)],
                                  u_bufs.at[slot, j], sems.at[slot]).wait()
            pltpu.make_async_copy(iw3_hbm.at[:, :, pl.ds(0, 128)],
                                  i_bufs.at[slot, j], sems.at[slot]).wait()

        sl = pl.ds(g * _G, _L)
        uc = idx_u[sl]
        ic = idx_i[sl]
        lu = uc & 127
        li = ic & 127
        slot_vec = jnp.full((_L,), slot, jnp.int32)
        acc = bu_v[sl] + bi_v[sl] + bias_vec
        for f in range(_LD):
            half = jnp.full((_L,), f // 8, jnp.int32)
            feat = jnp.full((_L,), f % 8, jnp.int32)
            uv = plsc.load_gather(u_bufs, [slot_vec, elem, half, feat, lu])
            iv = plsc.load_gather(i_bufs, [slot_vec, elem, half, feat, li])
            acc = acc + uv * iv
        out_v[sl] = acc
        return 0

    lax.fori_loop(0, _NGROUP, group, 0)
    pltpu.sync_copy(out_v.at[pl.ds(0, _BPW)], out_hbm.at[pl.ds(base, _BPW)])


@functools.partial(
    pl.kernel,
    out_type=jax.ShapeDtypeStruct((_BATCH,), jnp.float32),
    mesh=plsc.VectorSubcoreMesh(core_axis_name="c", subcore_axis_name="s"),
    scratch_types=[
        pltpu.VMEM((_PAD,), jnp.int32),
        pltpu.VMEM((_PAD,), jnp.int32),
        pltpu.VMEM((_PAD,), jnp.float32),
        pltpu.VMEM((_PAD,), jnp.float32),
        pltpu.VMEM((_L,), jnp.float32),
        pltpu.VMEM((_PAD,), jnp.float32),
        pltpu.VMEM((2, _G, 2, 8, 128), jnp.float32),
        pltpu.VMEM((2, _G, 2, 8, 128), jnp.float32),
        pltpu.SemaphoreType.DMA,
        pltpu.SemaphoreType.DMA((2,)),
    ],
    compiler_params=pltpu.CompilerParams(needs_layout_passes=False,
                                         use_tc_tiling_on_sc=True,
                                         disable_bounds_checks=True),
)
def _main_kernel(*args):
    _main_body(*args)


def kernel(train_x, user_weight, item_weight, bias_user_weight, bias_item_weight, bias):
    uid = train_x[:, 0]
    iid = train_x[:, 1]
    bias16 = jnp.broadcast_to(bias, (_L,))
    uw3 = user_weight.T.reshape(2, 8, _N)
    iw3 = item_weight.T.reshape(2, 8, _N)
    return _main_kernel(uid, iid, uw3, iw3, bias_user_weight.reshape(-1),
                        bias_item_weight.reshape(-1), bias16)


# speedup vs baseline: 5.7640x; 1.5711x over previous
import functools

import jax
import jax.numpy as jnp
from jax import lax
from jax.experimental import pallas as pl
from jax.experimental.pallas import tpu as pltpu, tpu_sc as plsc

_INFO = plsc.get_sparse_core_info()
_NC = _INFO.num_cores
_NS = _INFO.num_subcores
_NW = _NC * _NS
_L = _INFO.num_lanes

_N = 1000000
_BATCH = 16384
_LD = 16
_BPW = _BATCH // _NW
_CHUNK = 128
_NCHUNK = _BPW // _CHUNK
_G = 8
_NGROUP = _BPW // _G
_PAD = _BPW + _L


def _fire(g, slot, idx_u, idx_i, uw3_hbm, iw3_hbm, u_bufs, i_bufs, sems):
    uc = idx_u[pl.ds(g * _G, _L)]
    ic = idx_i[pl.ds(g * _G, _L)]
    ub = (uc >> 7) << 7
    ib = (ic >> 7) << 7
    for j in range(_G):
        cu = pl.multiple_of(ub[j], 128)
        ci = pl.multiple_of(ib[j], 128)
        pltpu.async_copy(uw3_hbm.at[:, :, pl.ds(cu, 128)],
                         u_bufs.at[slot, j], sems.at[slot])
        pltpu.async_copy(iw3_hbm.at[:, :, pl.ds(ci, 128)],
                         i_bufs.at[slot, j], sems.at[slot])


def _dots_body(uid_hbm, iid_hbm, uw3_hbm, iw3_hbm, out_hbm,
               idx_u, idx_i, out_v, u_bufs, i_bufs, sems):
    wid = lax.axis_index("s") * _NC + lax.axis_index("c")
    base = wid * _BPW
    pltpu.sync_copy(uid_hbm.at[pl.ds(base, _BPW)], idx_u.at[pl.ds(0, _BPW)])
    pltpu.sync_copy(iid_hbm.at[pl.ds(base, _BPW)], idx_i.at[pl.ds(0, _BPW)])

    _fire(0, 0, idx_u, idx_i, uw3_hbm, iw3_hbm, u_bufs, i_bufs, sems)

    lane = lax.iota(jnp.int32, _L)
    elem = lane & (_G - 1)
    zero = jnp.zeros((_L,), jnp.float32)

    def group(g, _):
        slot = g & 1

        @pl.when(g + 1 < _NGROUP)
        def _():
            _fire(g + 1, slot ^ 1, idx_u, idx_i, uw3_hbm, iw3_hbm,
                  u_bufs, i_bufs, sems)

        for j in range(_G):
            pltpu.make_async_copy(uw3_hbm.at[:, :, pl.ds(0, 128)],
                                  u_bufs.at[slot, j], sems.at[slot]).wait()
            pltpu.make_async_copy(iw3_hbm.at[:, :, pl.ds(0, 128)],
                                  i_bufs.at[slot, j], sems.at[slot]).wait()

        sl = pl.ds(g * _G, _L)
        uc = idx_u[sl]
        ic = idx_i[sl]
        lu = uc & 127
        li = ic & 127
        slot_vec = jnp.full((_L,), slot, jnp.int32)
        acc = zero
        for f in range(_LD):
            half = jnp.full((_L,), f // 8, jnp.int32)
            feat = jnp.full((_L,), f % 8, jnp.int32)
            uv = plsc.load_gather(u_bufs, [slot_vec, elem, half, feat, lu])
            iv = plsc.load_gather(i_bufs, [slot_vec, elem, half, feat, li])
            acc = acc + uv * iv
        out_v[sl] = acc
        return 0

    lax.fori_loop(0, _NGROUP, group, 0)
    pltpu.sync_copy(out_v.at[pl.ds(0, _BPW)], out_hbm.at[pl.ds(base, _BPW)])


@functools.partial(
    pl.kernel,
    out_type=jax.ShapeDtypeStruct((_BATCH,), jnp.float32),
    mesh=plsc.VectorSubcoreMesh(core_axis_name="c", subcore_axis_name="s"),
    scratch_types=[
        pltpu.VMEM((_PAD,), jnp.int32),
        pltpu.VMEM((_PAD,), jnp.int32),
        pltpu.VMEM((_PAD,), jnp.float32),
        pltpu.VMEM((2, _G, 2, 8, 128), jnp.float32),
        pltpu.VMEM((2, _G, 2, 8, 128), jnp.float32),
        pltpu.SemaphoreType.DMA((2,)),
    ],
    compiler_params=pltpu.CompilerParams(needs_layout_passes=False,
                                         use_tc_tiling_on_sc=True,
                                         disable_bounds_checks=True),
)
def _dots_kernel(*args):
    _dots_body(*args)


def _bias_body(uid_hbm, iid_hbm, bu_hbm, bi_hbm, bias_hbm, dots_hbm, out_hbm,
               idx_u, idx_i, bu_v, bi_v, bias_v, dots_v, out_v, sem):
    wid = lax.axis_index("s") * _NC + lax.axis_index("c")
    base = wid * _BPW
    pltpu.sync_copy(uid_hbm.at[pl.ds(base, _BPW)], idx_u)
    pltpu.sync_copy(iid_hbm.at[pl.ds(base, _BPW)], idx_i)
    pltpu.sync_copy(dots_hbm.at[pl.ds(base, _BPW)], dots_v)
    pltpu.sync_copy(bias_hbm, bias_v)
    copies = []
    for j in range(_NCHUNK):
        sl = pl.ds(j * _CHUNK, _CHUNK)
        copies.append(pltpu.async_copy(bu_hbm.at[idx_u.at[sl]], bu_v.at[sl], sem))
        copies.append(pltpu.async_copy(bi_hbm.at[idx_i.at[sl]], bi_v.at[sl], sem))
    for cp in copies:
        cp.wait()
    bias_vec = bias_v[...]

    def group(g, _):
        sl = pl.ds(g * _L, _L)
        out_v[sl] = dots_v[sl] + bu_v[sl] + bi_v[sl] + bias_vec
        return 0

    lax.fori_loop(0, _BPW // _L, group, 0)
    pltpu.sync_copy(out_v, out_hbm.at[pl.ds(base, _BPW)])


@functools.partial(
    pl.kernel,
    out_type=jax.ShapeDtypeStruct((_BATCH,), jnp.float32),
    mesh=plsc.VectorSubcoreMesh(core_axis_name="c", subcore_axis_name="s"),
    scratch_types=[
        pltpu.VMEM((_BPW,), jnp.int32),
        pltpu.VMEM((_BPW,), jnp.int32),
        pltpu.VMEM((_BPW,), jnp.float32),
        pltpu.VMEM((_BPW,), jnp.float32),
        pltpu.VMEM((_L,), jnp.float32),
        pltpu.VMEM((_BPW,), jnp.float32),
        pltpu.VMEM((_BPW,), jnp.float32),
        pltpu.SemaphoreType.DMA,
    ],
    compiler_params=pltpu.CompilerParams(needs_layout_passes=False,
                                         use_tc_tiling_on_sc=False),
)
def _bias_kernel(*args):
    _bias_body(*args)


def kernel(train_x, user_weight, item_weight, bias_user_weight, bias_item_weight, bias):
    uid = train_x[:, 0]
    iid = train_x[:, 1]
    bias16 = jnp.broadcast_to(bias, (_L,))
    uw3 = user_weight.T.reshape(2, 8, _N)
    iw3 = item_weight.T.reshape(2, 8, _N)
    dots = _dots_kernel(uid, iid, uw3, iw3)
    return _bias_kernel(uid, iid, bias_user_weight.reshape(-1),
                        bias_item_weight.reshape(-1), bias16, dots)
